# baseline (speedup 1.0000x reference)
import functools

import jax
import jax.numpy as jnp
from jax import lax
from jax.experimental import pallas as pl
from jax.experimental.pallas import tpu as pltpu
from jax.experimental.pallas import tpu_sc as plsc

N_NODES = 10000
D = 128
G = 64
L = 16
NC = 2
NS = 16
NW = NC * NS
ROWS_PER = 320
ROWS_LAST = N_NODES - (NW - 1) * ROWS_PER
CHUNK = 80
N_CHUNKS = ROWS_PER // CHUNK


def _sc_segment_sum(x, batch):
    mesh = plsc.VectorSubcoreMesh(core_axis_name="c", subcore_axis_name="s")

    @functools.partial(
        pl.kernel,
        mesh=mesh,
        out_type=jax.ShapeDtypeStruct((NC, 4 * G, D), jnp.float32),
        scratch_types=[
            pltpu.VMEM((ROWS_PER, D), jnp.float32),
            pltpu.VMEM((N_CHUNKS, CHUNK), jnp.int32),
            pltpu.VMEM((16, D), jnp.float32),
            pltpu.VMEM_SHARED((4 * G, D), jnp.float32),
            pltpu.SemaphoreType.DMA,
            pltpu.SemaphoreType.DMA,
            pltpu.SemaphoreType.DMA,
            pltpu.SemaphoreType.DMA,
            pltpu.SemaphoreType.DMA,
        ],
    )
    def seg_sum(x_hbm, b_hbm, out_hbm,
                xbuf, idxbuf, zbuf, acc,
                sem_g0, sem_g1, sem_g2, sem_g3, sem_idx):
        c = lax.axis_index("c")
        s = lax.axis_index("s")
        wid = s * NC + c
        base = wid * ROWS_PER
        gsems = [sem_g0, sem_g1, sem_g2, sem_g3]
        nch = jnp.where(wid == NW - 1, 1, N_CHUNKS)

        idx_cps = []
        gathers = []
        for j in range(N_CHUNKS):
            bj = jnp.minimum(base + j * CHUNK, N_NODES - CHUNK)
            idx_cps.append(pltpu.async_copy(
                b_hbm.at[pl.ds(bj, CHUNK)], idxbuf.at[j], sem_idx))
            gathers.append(pltpu.async_copy(
                x_hbm.at[pl.ds(bj, CHUNK)],
                xbuf.at[pl.ds(j * CHUNK, CHUNK)], gsems[j]))

        zero16 = jnp.zeros((L,), jnp.float32)
        for r in range(16):
            for k in range(D // L):
                zbuf[r, pl.ds(k * L, L)] = zero16
        pltpu.sync_copy(zbuf, acc.at[pl.ds(s * 16, 16)])

        plsc.subcore_barrier()

        for cp in idx_cps:
            cp.wait()
        roff = jnp.broadcast_to((s % 4) * G, (L,)).astype(jnp.int32)
        for j in range(N_CHUNKS):
            for t in range(CHUNK // L):
                v = idxbuf[j, pl.ds(t * L, L)]
                idxbuf[j, pl.ds(t * L, L)] = v + roff
        for j in range(N_CHUNKS):
            gathers[j].wait()

            @pl.when(j < nch)
            def _(j=j):
                pltpu.sync_copy(xbuf.at[pl.ds(j * CHUNK, CHUNK)],
                                acc.at[idxbuf.at[j]], add=True)

        plsc.subcore_barrier()

        pltpu.sync_copy(acc.at[pl.ds(s * 16, 16)],
                        out_hbm.at[c].at[pl.ds(s * 16, 16)])

    return seg_sum(x, batch)


def _tc_mlp_pre(u, W1, b1):
    def body(u_ref, w1a_ref, b1_ref, o_ref):
        o_ref[...] = (jnp.dot(u_ref[...], w1a_ref[...],
                              preferred_element_type=jnp.float32)
                      + b1_ref[...])

    return pl.pallas_call(
        body,
        grid=(1,),
        out_shape=jax.ShapeDtypeStruct((G, 256), jnp.float32),
        in_specs=[
            pl.BlockSpec((G, 128), lambda i: (0, 0)),
            pl.BlockSpec((128, 256), lambda i: (0, 0)),
            pl.BlockSpec((1, 256), lambda i: (0, 0)),
        ],
        out_specs=pl.BlockSpec((G, 256), lambda i: (0, 0)),
    )(u, W1, b1.reshape(1, -1))


def _tc_mlp_post(partials, pre, W1, W2, b2):
    def body(p_ref, pre_ref, w1b_ref, w2_ref, b2_ref, o_ref):
        pooled = jnp.sum(p_ref[...].reshape(2 * 4, G, D), axis=0)
        h = pre_ref[...] + jnp.dot(pooled, w1b_ref[...],
                                   preferred_element_type=jnp.float32)
        h = jnp.maximum(h, 0.0)
        o_ref[...] = (jnp.dot(h, w2_ref[...], preferred_element_type=jnp.float32)
                      + b2_ref[...])

    return pl.pallas_call(
        body,
        grid=(1,),
        out_shape=jax.ShapeDtypeStruct((G, 128), jnp.float32),
        in_specs=[
            pl.BlockSpec((NC, 4 * G, D), lambda i: (0, 0, 0)),
            pl.BlockSpec((G, 256), lambda i: (0, 0)),
            pl.BlockSpec((128, 256), lambda i: (1, 0)),
            pl.BlockSpec((256, 128), lambda i: (0, 0)),
            pl.BlockSpec((1, 128), lambda i: (0, 0)),
        ],
        out_specs=pl.BlockSpec((G, 128), lambda i: (0, 0)),
    )(partials, pre, W1, W2, b2.reshape(1, -1))


def kernel(x, edge_index, edge_attr, u, batch, W1, b1, W2, b2):
    pre = _tc_mlp_pre(u, W1, b1)
    partials = _sc_segment_sum(x, batch)
    return _tc_mlp_post(partials, pre, W1, W2, b2)


# speedup vs baseline: 1.0875x; 1.0875x over previous
import functools

import jax
import jax.numpy as jnp
from jax import lax
from jax.experimental import pallas as pl
from jax.experimental.pallas import tpu as pltpu
from jax.experimental.pallas import tpu_sc as plsc

N_NODES = 10000
N_TC = 5000
N_SC = N_NODES - N_TC
D = 128
G = 64
L = 16
NC = 2
NS = 16
NW = NC * NS
ROWS_PER = 160
CHUNK = 40
N_CHUNKS = ROWS_PER // CHUNK


def _sc_segment_sum(x, batch):
    mesh = plsc.VectorSubcoreMesh(core_axis_name="c", subcore_axis_name="s")

    @functools.partial(
        pl.kernel,
        mesh=mesh,
        out_type=jax.ShapeDtypeStruct((NC, G, D), jnp.float32),
        scratch_types=[
            pltpu.VMEM((ROWS_PER, D), jnp.float32),
            pltpu.VMEM((N_CHUNKS, CHUNK), jnp.int32),
            pltpu.VMEM((8, D), jnp.float32),
            pltpu.VMEM_SHARED((G, D), jnp.float32),
            pltpu.SemaphoreType.DMA,
            pltpu.SemaphoreType.DMA,
            pltpu.SemaphoreType.DMA,
            pltpu.SemaphoreType.DMA,
            pltpu.SemaphoreType.DMA,
        ],
    )
    def seg_sum(x_hbm, b_hbm, out_hbm,
                xbuf, idxbuf, zbuf, acc,
                sem_g0, sem_g1, sem_g2, sem_g3, sem_idx):
        c = lax.axis_index("c")
        s = lax.axis_index("s")
        wid = s * NC + c
        base = N_TC + wid * ROWS_PER
        gsems = [sem_g0, sem_g1, sem_g2, sem_g3]
        nch = jnp.where(wid == NW - 1, 1, N_CHUNKS)

        idx_cps = []
        gathers = []
        for j in range(N_CHUNKS):
            bj = jnp.minimum(base + j * CHUNK, N_NODES - CHUNK)
            idx_cps.append(pltpu.async_copy(
                b_hbm.at[pl.ds(bj, CHUNK)], idxbuf.at[j], sem_idx))
            gathers.append(pltpu.async_copy(
                x_hbm.at[pl.ds(bj, CHUNK)],
                xbuf.at[pl.ds(j * CHUNK, CHUNK)], gsems[j]))

        zero16 = jnp.zeros((L,), jnp.float32)

        @pl.when(s < 8)
        def _():
            for r in range(8):
                for k in range(D // L):
                    zbuf[r, pl.ds(k * L, L)] = zero16
            pltpu.sync_copy(zbuf, acc.at[pl.ds(s * 8, 8)])

        plsc.subcore_barrier()

        for cp in idx_cps:
            cp.wait()
        for j in range(N_CHUNKS):
            gathers[j].wait()

            @pl.when(j < nch)
            def _(j=j):
                pltpu.sync_copy(xbuf.at[pl.ds(j * CHUNK, CHUNK)],
                                acc.at[idxbuf.at[j]], add=True)

        plsc.subcore_barrier()

        @pl.when(s < 8)
        def _():
            pltpu.sync_copy(acc.at[pl.ds(s * 8, 8)],
                            out_hbm.at[c].at[pl.ds(s * 8, 8)])

    return seg_sum(x, batch)


def _tc_half_pool(x, batch2d):
    def body(x_ref, b_ref, o_ref):
        ids = lax.slice(b_ref[...], (0, 0), (1, N_TC))
        seg = lax.broadcasted_iota(jnp.int32, (G, 1), 0)
        mask = (ids == seg).astype(jnp.float32)
        o_ref[...] = jnp.dot(mask, x_ref[...],
                             preferred_element_type=jnp.float32)

    return pl.pallas_call(
        body,
        grid=(1,),
        out_shape=jax.ShapeDtypeStruct((G, D), jnp.float32),
        in_specs=[
            pl.BlockSpec((N_TC, D), lambda i: (0, 0)),
            pl.BlockSpec((1, N_NODES), lambda i: (0, 0)),
        ],
        out_specs=pl.BlockSpec((G, D), lambda i: (0, 0)),
    )(x, batch2d)


def _tc_mlp_pre(u, W1, b1):
    def body(u_ref, w1a_ref, b1_ref, o_ref):
        o_ref[...] = (jnp.dot(u_ref[...], w1a_ref[...],
                              preferred_element_type=jnp.float32)
                      + b1_ref[...])

    return pl.pallas_call(
        body,
        grid=(1,),
        out_shape=jax.ShapeDtypeStruct((G, 256), jnp.float32),
        in_specs=[
            pl.BlockSpec((G, 128), lambda i: (0, 0)),
            pl.BlockSpec((128, 256), lambda i: (0, 0)),
            pl.BlockSpec((1, 256), lambda i: (0, 0)),
        ],
        out_specs=pl.BlockSpec((G, 256), lambda i: (0, 0)),
    )(u, W1, b1.reshape(1, -1))


def _tc_mlp_post(partials, pooled_tc, pre, W1, W2, b2):
    def body(p_ref, ptc_ref, pre_ref, w1b_ref, w2_ref, b2_ref, o_ref):
        pooled = p_ref[0] + p_ref[1] + ptc_ref[...]
        h = pre_ref[...] + jnp.dot(pooled, w1b_ref[...],
                                   preferred_element_type=jnp.float32)
        h = jnp.maximum(h, 0.0)
        o_ref[...] = (jnp.dot(h, w2_ref[...], preferred_element_type=jnp.float32)
                      + b2_ref[...])

    return pl.pallas_call(
        body,
        grid=(1,),
        out_shape=jax.ShapeDtypeStruct((G, 128), jnp.float32),
        in_specs=[
            pl.BlockSpec((NC, G, D), lambda i: (0, 0, 0)),
            pl.BlockSpec((G, D), lambda i: (0, 0)),
            pl.BlockSpec((G, 256), lambda i: (0, 0)),
            pl.BlockSpec((128, 256), lambda i: (1, 0)),
            pl.BlockSpec((256, 128), lambda i: (0, 0)),
            pl.BlockSpec((1, 128), lambda i: (0, 0)),
        ],
        out_specs=pl.BlockSpec((G, 128), lambda i: (0, 0)),
    )(partials, pooled_tc, pre, W1, W2, b2.reshape(1, -1))


def kernel(x, edge_index, edge_attr, u, batch, W1, b1, W2, b2):
    batch2d = batch.reshape(1, N_NODES)
    pooled_tc = _tc_half_pool(x, batch2d)
    pre = _tc_mlp_pre(u, W1, b1)
    partials = _sc_segment_sum(x, batch)
    return _tc_mlp_post(partials, pooled_tc, pre, W1, W2, b2)


# speedup vs baseline: 1.1089x; 1.0197x over previous
import functools

import jax
import jax.numpy as jnp
from jax import lax
from jax.experimental import pallas as pl
from jax.experimental.pallas import tpu as pltpu
from jax.experimental.pallas import tpu_sc as plsc

N_NODES = 10000
N_TC = 6240
N_SC = N_NODES - N_TC
D = 128
G = 64
L = 16
NC = 2
NS = 16
NW = NC * NS
ROWS_PER = 120
CHUNK = 40
N_CHUNKS = ROWS_PER // CHUNK


def _sc_segment_sum(x, batch):
    mesh = plsc.VectorSubcoreMesh(core_axis_name="c", subcore_axis_name="s")

    @functools.partial(
        pl.kernel,
        mesh=mesh,
        out_type=jax.ShapeDtypeStruct((NC, G, D), jnp.float32),
        scratch_types=[
            pltpu.VMEM((ROWS_PER, D), jnp.float32),
            pltpu.VMEM((N_CHUNKS, CHUNK), jnp.int32),
            pltpu.VMEM((8, D), jnp.float32),
            pltpu.VMEM_SHARED((G, D), jnp.float32),
            pltpu.SemaphoreType.DMA,
            pltpu.SemaphoreType.DMA,
            pltpu.SemaphoreType.DMA,
            pltpu.SemaphoreType.DMA,
            pltpu.SemaphoreType.DMA,
        ],
    )
    def seg_sum(x_hbm, b_hbm, out_hbm,
                xbuf, idxbuf, zbuf, acc,
                sem_g0, sem_g1, sem_g2, sem_g3, sem_idx):
        c = lax.axis_index("c")
        s = lax.axis_index("s")
        wid = s * NC + c
        base = N_TC + wid * ROWS_PER
        gsems = [sem_g0, sem_g1, sem_g2, sem_g3]
        nch = jnp.where(wid == NW - 1, 1, N_CHUNKS)

        idx_cps = []
        gathers = []
        for j in range(N_CHUNKS):
            bj = jnp.minimum(base + j * CHUNK, N_NODES - CHUNK)
            idx_cps.append(pltpu.async_copy(
                b_hbm.at[pl.ds(bj, CHUNK)], idxbuf.at[j], sem_idx))
            gathers.append(pltpu.async_copy(
                x_hbm.at[pl.ds(bj, CHUNK)],
                xbuf.at[pl.ds(j * CHUNK, CHUNK)], gsems[j]))

        zero16 = jnp.zeros((L,), jnp.float32)

        @pl.when(s < 8)
        def _():
            for r in range(8):
                for k in range(D // L):
                    zbuf[r, pl.ds(k * L, L)] = zero16
            pltpu.sync_copy(zbuf, acc.at[pl.ds(s * 8, 8)])

        plsc.subcore_barrier()

        for cp in idx_cps:
            cp.wait()
        for j in range(N_CHUNKS):
            gathers[j].wait()

            @pl.when(j < nch)
            def _(j=j):
                pltpu.sync_copy(xbuf.at[pl.ds(j * CHUNK, CHUNK)],
                                acc.at[idxbuf.at[j]], add=True)

        plsc.subcore_barrier()

        @pl.when(s < 8)
        def _():
            pltpu.sync_copy(acc.at[pl.ds(s * 8, 8)],
                            out_hbm.at[c].at[pl.ds(s * 8, 8)])

    return seg_sum(x, batch)


def _tc_half_pool(x, batch2d):
    def body(x_ref, b_ref, o_ref):
        ids = lax.slice(b_ref[...], (0, 0), (1, N_TC))
        seg = lax.broadcasted_iota(jnp.int32, (G, 1), 0)
        mask = (ids == seg).astype(jnp.float32)
        o_ref[...] = jnp.dot(mask, x_ref[...],
                             preferred_element_type=jnp.float32)

    return pl.pallas_call(
        body,
        grid=(1,),
        out_shape=jax.ShapeDtypeStruct((G, D), jnp.float32),
        in_specs=[
            pl.BlockSpec((N_TC, D), lambda i: (0, 0)),
            pl.BlockSpec((1, N_NODES), lambda i: (0, 0)),
        ],
        out_specs=pl.BlockSpec((G, D), lambda i: (0, 0)),
    )(x, batch2d)


def _tc_mlp_pre(u, W1, b1):
    def body(u_ref, w1a_ref, b1_ref, o_ref):
        o_ref[...] = (jnp.dot(u_ref[...], w1a_ref[...],
                              preferred_element_type=jnp.float32)
                      + b1_ref[...])

    return pl.pallas_call(
        body,
        grid=(1,),
        out_shape=jax.ShapeDtypeStruct((G, 256), jnp.float32),
        in_specs=[
            pl.BlockSpec((G, 128), lambda i: (0, 0)),
            pl.BlockSpec((128, 256), lambda i: (0, 0)),
            pl.BlockSpec((1, 256), lambda i: (0, 0)),
        ],
        out_specs=pl.BlockSpec((G, 256), lambda i: (0, 0)),
    )(u, W1, b1.reshape(1, -1))


def _tc_mlp_post(partials, pooled_tc, pre, W1, W2, b2):
    def body(p_ref, ptc_ref, pre_ref, w1b_ref, w2_ref, b2_ref, o_ref):
        pooled = p_ref[0] + p_ref[1] + ptc_ref[...]
        h = pre_ref[...] + jnp.dot(pooled, w1b_ref[...],
                                   preferred_element_type=jnp.float32)
        h = jnp.maximum(h, 0.0)
        o_ref[...] = (jnp.dot(h, w2_ref[...], preferred_element_type=jnp.float32)
                      + b2_ref[...])

    return pl.pallas_call(
        body,
        grid=(1,),
        out_shape=jax.ShapeDtypeStruct((G, 128), jnp.float32),
        in_specs=[
            pl.BlockSpec((NC, G, D), lambda i: (0, 0, 0)),
            pl.BlockSpec((G, D), lambda i: (0, 0)),
            pl.BlockSpec((G, 256), lambda i: (0, 0)),
            pl.BlockSpec((128, 256), lambda i: (1, 0)),
            pl.BlockSpec((256, 128), lambda i: (0, 0)),
            pl.BlockSpec((1, 128), lambda i: (0, 0)),
        ],
        out_specs=pl.BlockSpec((G, 128), lambda i: (0, 0)),
    )(partials, pooled_tc, pre, W1, W2, b2.reshape(1, -1))


def kernel(x, edge_index, edge_attr, u, batch, W1, b1, W2, b2):
    batch2d = batch.reshape(1, N_NODES)
    pooled_tc = _tc_half_pool(x, batch2d)
    pre = _tc_mlp_pre(u, W1, b1)
    partials = _sc_segment_sum(x, batch)
    return _tc_mlp_post(partials, pooled_tc, pre, W1, W2, b2)


# speedup vs baseline: 1.1211x; 1.0110x over previous
import functools

import jax
import jax.numpy as jnp
from jax import lax
from jax.experimental import pallas as pl
from jax.experimental.pallas import tpu as pltpu
from jax.experimental.pallas import tpu_sc as plsc

N_NODES = 10000
N_TC = 6992
N_SC = N_NODES - N_TC
D = 128
G = 64
L = 16
NC = 2
NS = 16
NW = NC * NS
ROWS_PER = 96
CHUNK = 32
N_CHUNKS = ROWS_PER // CHUNK


def _sc_segment_sum(x, batch):
    mesh = plsc.VectorSubcoreMesh(core_axis_name="c", subcore_axis_name="s")

    @functools.partial(
        pl.kernel,
        mesh=mesh,
        out_type=jax.ShapeDtypeStruct((NC, G, D), jnp.float32),
        scratch_types=[
            pltpu.VMEM((ROWS_PER, D), jnp.float32),
            pltpu.VMEM((N_CHUNKS, CHUNK), jnp.int32),
            pltpu.VMEM((8, D), jnp.float32),
            pltpu.VMEM_SHARED((G, D), jnp.float32),
            pltpu.SemaphoreType.DMA,
            pltpu.SemaphoreType.DMA,
            pltpu.SemaphoreType.DMA,
            pltpu.SemaphoreType.DMA,
            pltpu.SemaphoreType.DMA,
        ],
    )
    def seg_sum(x_hbm, b_hbm, out_hbm,
                xbuf, idxbuf, zbuf, acc,
                sem_g0, sem_g1, sem_g2, sem_g3, sem_idx):
        c = lax.axis_index("c")
        s = lax.axis_index("s")
        wid = s * NC + c
        base = N_TC + wid * ROWS_PER
        gsems = [sem_g0, sem_g1, sem_g2, sem_g3]
        nch = jnp.where(wid == NW - 1, 1, N_CHUNKS)

        idx_cps = []
        gathers = []
        for j in range(N_CHUNKS):
            bj = jnp.minimum(base + j * CHUNK, N_NODES - CHUNK)
            idx_cps.append(pltpu.async_copy(
                b_hbm.at[pl.ds(bj, CHUNK)], idxbuf.at[j], sem_idx))
            gathers.append(pltpu.async_copy(
                x_hbm.at[pl.ds(bj, CHUNK)],
                xbuf.at[pl.ds(j * CHUNK, CHUNK)], gsems[j]))

        zero16 = jnp.zeros((L,), jnp.float32)

        @pl.when(s < 8)
        def _():
            for r in range(8):
                for k in range(D // L):
                    zbuf[r, pl.ds(k * L, L)] = zero16
            pltpu.sync_copy(zbuf, acc.at[pl.ds(s * 8, 8)])

        plsc.subcore_barrier()

        for cp in idx_cps:
            cp.wait()
        for j in range(N_CHUNKS):
            gathers[j].wait()

            @pl.when(j < nch)
            def _(j=j):
                pltpu.sync_copy(xbuf.at[pl.ds(j * CHUNK, CHUNK)],
                                acc.at[idxbuf.at[j]], add=True)

        plsc.subcore_barrier()

        @pl.when(s < 8)
        def _():
            pltpu.sync_copy(acc.at[pl.ds(s * 8, 8)],
                            out_hbm.at[c].at[pl.ds(s * 8, 8)])

    return seg_sum(x, batch)


def _tc_overlap(x, batch2d, u, W1, b1):
    def body(x_ref, b_ref, u_ref, w1a_ref, b1_ref, o_ref, pre_ref):
        ids = lax.slice(b_ref[...], (0, 0), (1, N_TC))
        seg = lax.broadcasted_iota(jnp.int32, (G, 1), 0)
        mask = (ids == seg).astype(jnp.float32)
        o_ref[...] = jnp.dot(mask, x_ref[...],
                             preferred_element_type=jnp.float32)
        pre_ref[...] = (jnp.dot(u_ref[...], w1a_ref[...],
                                preferred_element_type=jnp.float32)
                        + b1_ref[...])

    return pl.pallas_call(
        body,
        grid=(1,),
        out_shape=[jax.ShapeDtypeStruct((G, D), jnp.float32),
                   jax.ShapeDtypeStruct((G, 256), jnp.float32)],
        in_specs=[
            pl.BlockSpec((N_TC, D), lambda i: (0, 0)),
            pl.BlockSpec((1, N_NODES), lambda i: (0, 0)),
            pl.BlockSpec((G, 128), lambda i: (0, 0)),
            pl.BlockSpec((128, 256), lambda i: (0, 0)),
            pl.BlockSpec((1, 256), lambda i: (0, 0)),
        ],
        out_specs=[pl.BlockSpec((G, D), lambda i: (0, 0)),
                   pl.BlockSpec((G, 256), lambda i: (0, 0))],
    )(x, batch2d, u, W1, b1.reshape(1, -1))


def _tc_mlp_post(partials, pooled_tc, pre, W1, W2, b2):
    def body(p_ref, ptc_ref, pre_ref, w1b_ref, w2_ref, b2_ref, o_ref):
        pooled = p_ref[0] + p_ref[1] + ptc_ref[...]
        h = pre_ref[...] + jnp.dot(pooled, w1b_ref[...],
                                   preferred_element_type=jnp.float32)
        h = jnp.maximum(h, 0.0)
        o_ref[...] = (jnp.dot(h, w2_ref[...], preferred_element_type=jnp.float32)
                      + b2_ref[...])

    return pl.pallas_call(
        body,
        grid=(1,),
        out_shape=jax.ShapeDtypeStruct((G, 128), jnp.float32),
        in_specs=[
            pl.BlockSpec((NC, G, D), lambda i: (0, 0, 0)),
            pl.BlockSpec((G, D), lambda i: (0, 0)),
            pl.BlockSpec((G, 256), lambda i: (0, 0)),
            pl.BlockSpec((128, 256), lambda i: (1, 0)),
            pl.BlockSpec((256, 128), lambda i: (0, 0)),
            pl.BlockSpec((1, 128), lambda i: (0, 0)),
        ],
        out_specs=pl.BlockSpec((G, 128), lambda i: (0, 0)),
    )(partials, pooled_tc, pre, W1, W2, b2.reshape(1, -1))


def kernel(x, edge_index, edge_attr, u, batch, W1, b1, W2, b2):
    batch2d = batch.reshape(1, N_NODES)
    pooled_tc, pre = _tc_overlap(x, batch2d, u, W1, b1)
    partials = _sc_segment_sum(x, batch)
    return _tc_mlp_post(partials, pooled_tc, pre, W1, W2, b2)
